# baseline (speedup 1.0000x reference)
import jax
import jax.numpy as jnp
from jax import lax
from jax.experimental import pallas as pl
from jax.experimental.pallas import tpu as pltpu
from jax.experimental.pallas import tpu_sc as plsc

VOCAB = 1000000
DIM = 64
D2 = DIM // 2
PD = 128
QR = 4
B = 16384
NNEG = 20

NC = 2
NS = 16
L = 16
NW = NC * NS
BPW = B // NW
CH = 16
NCH = BPW // CH
CNC = CH * NNEG
CN_SPLIT = ((0, 128), (128, 128), (256, 64))


def _fire_chunk(tabs, idxs, bufs, sem, c):
    t_tab, c_tab = tabs
    idx_tq, idx_cpq, idx_cnq = idxs
    t_rows, cp_rows, cn_rows = bufs
    pltpu.async_copy(t_tab.at[idx_tq.at[pl.ds(c * CH, CH)]], t_rows, sem)
    pltpu.async_copy(c_tab.at[idx_cpq.at[pl.ds(c * CH, CH)]], cp_rows, sem)
    for off, n in CN_SPLIT:
        pltpu.async_copy(c_tab.at[idx_cnq.at[pl.ds(c * CNC + off, n)]],
                         cn_rows.at[pl.ds(off, n)], sem)


def _drain_chunk(tabs, idxs, bufs, sem, c):
    t_tab, c_tab = tabs
    idx_tq, idx_cpq, idx_cnq = idxs
    t_rows, cp_rows, cn_rows = bufs
    pltpu.make_async_copy(t_tab.at[idx_tq.at[pl.ds(c * CH, CH)]], t_rows, sem).wait()
    pltpu.make_async_copy(c_tab.at[idx_cpq.at[pl.ds(c * CH, CH)]], cp_rows, sem).wait()
    for off, n in CN_SPLIT:
        pltpu.make_async_copy(c_tab.at[idx_cnq.at[pl.ds(c * CNC + off, n)]],
                              cn_rows.at[pl.ds(off, n)], sem).wait()


def _unpack2(w):
    return plsc.unpack(plsc.bitcast(w, jnp.bfloat16),
                       format=plsc.PackFormat.INTERLEAVED)


def _sc_scores_body(t_tab, c_tab, t_idx, cp_idx, cn_idx, out,
                    idx_t, idx_cp, idx_cn, idx_tq, idx_cpq, idx_cnq,
                    t_rows, cp_rows, cn_rows, sp_out, sn_out, sem0, sem1):
    wid = lax.axis_index("s") * NC + lax.axis_index("c")
    base = wid * BPW

    pltpu.sync_copy(t_idx.at[pl.ds(base, BPW)], idx_t)
    pltpu.sync_copy(cp_idx.at[pl.ds(base, BPW)], idx_cp)
    pltpu.sync_copy(cn_idx.at[pl.ds(base * NNEG, BPW * NNEG)], idx_cn)

    def shift_fill(src, dst, count):
        def step(i, _):
            v = src[pl.ds(i * L, L)]
            dst[pl.ds(i * L, L)] = lax.shift_right_logical(v, 2)
            return 0
        lax.fori_loop(0, count, step, 0, unroll=8)

    shift_fill(idx_t, idx_tq, BPW // L)
    shift_fill(idx_cp, idx_cpq, BPW // L)
    shift_fill(idx_cn, idx_cnq, BPW * NNEG // L)

    tabs = (t_tab, c_tab)
    idxs = (idx_tq, idx_cpq, idx_cnq)
    bufs = [(t_rows.at[k], cp_rows.at[k], cn_rows.at[k]) for k in (0, 1)]
    sems = (sem0, sem1)

    lane = lax.iota(jnp.int32, L)
    lane20 = lane * NNEG

    def compute(c, k):
        par_t = (plsc.load_gather(idx_t, [c * CH + lane]) & 3) << 5
        par_cp = (plsc.load_gather(idx_cp, [c * CH + lane]) & 3) << 5
        par_cn = [
            (plsc.load_gather(idx_cn, [c * CNC + n + lane20]) & 3) << 5
            for n in range(NNEG)
        ]
        cn_row = [lane20 + n for n in range(NNEG)]

        def dot_step(d, carry):
            s_p, s_n = carry
            dv = (lane + d) & (D2 - 1)
            ta, tb = _unpack2(plsc.load_gather(t_rows.at[k], [lane, par_t + dv]))
            ca, cb = _unpack2(plsc.load_gather(cp_rows.at[k], [lane, par_cp + dv]))
            na, nb = _unpack2(plsc.load_gather(cn_rows.at[k], [cn_row[0], par_cn[0] + dv]))
            for n in range(1, NNEG):
                xa, xb = _unpack2(
                    plsc.load_gather(cn_rows.at[k], [cn_row[n], par_cn[n] + dv]))
                na = na + xa
                nb = nb + xb
            s_p = s_p + ta * ca + tb * cb
            s_n = s_n + ta * na + tb * nb
            return s_p, s_n

        zero = jnp.zeros((L,), jnp.float32)
        s_p, s_n = lax.fori_loop(0, D2, dot_step, (zero, zero), unroll=4)
        sp_out[pl.ds(c * CH, L)] = -s_p
        sn_out[pl.ds(c * CH, L)] = s_n

    _fire_chunk(tabs, idxs, bufs[0], sems[0], 0)

    def loop_body(i, _):
        c0 = 2 * i
        _fire_chunk(tabs, idxs, bufs[1], sems[1], c0 + 1)
        _drain_chunk(tabs, idxs, bufs[0], sems[0], c0)
        compute(c0, 0)

        @pl.when(c0 + 2 < NCH)
        def _():
            _fire_chunk(tabs, idxs, bufs[0], sems[0], c0 + 2)
        _drain_chunk(tabs, idxs, bufs[1], sems[1], c0 + 1)
        compute(c0 + 1, 1)
        return 0

    lax.fori_loop(0, NCH // 2, loop_body, 0)

    pltpu.sync_copy(sp_out, out.at[pl.ds(base, BPW)])
    pltpu.sync_copy(sn_out, out.at[pl.ds(B + base, BPW)])


def _sc_scores(t_tab, c_tab, t_idx, cp_idx, cn_idx):
    mesh = plsc.VectorSubcoreMesh(core_axis_name="c", subcore_axis_name="s")
    return pl.kernel(
        _sc_scores_body,
        out_type=jax.ShapeDtypeStruct((2 * B,), jnp.float32),
        mesh=mesh,
        compiler_params=pltpu.CompilerParams(needs_layout_passes=False),
        scratch_types=[
            pltpu.VMEM((BPW,), jnp.int32),
            pltpu.VMEM((BPW,), jnp.int32),
            pltpu.VMEM((BPW * NNEG,), jnp.int32),
            pltpu.VMEM((BPW,), jnp.int32),
            pltpu.VMEM((BPW,), jnp.int32),
            pltpu.VMEM((BPW * NNEG,), jnp.int32),
            pltpu.VMEM((2, CH, PD), jnp.int32),
            pltpu.VMEM((2, CH, PD), jnp.int32),
            pltpu.VMEM((2, CNC, PD), jnp.int32),
            pltpu.VMEM((BPW,), jnp.float32),
            pltpu.VMEM((BPW,), jnp.float32),
            pltpu.SemaphoreType.DMA,
            pltpu.SemaphoreType.DMA,
        ],
    )(t_tab, c_tab, t_idx, cp_idx, cn_idx)


def _loss_body(s_ref, o_ref):
    x = s_ref[...]
    z = jnp.minimum(x, 0.0) - jnp.log(1.0 + jnp.exp(-jnp.abs(x)))
    o_ref[0, 0] = jnp.sum(z) * (1.0 / B)


def _tc_loss(scores):
    out = pl.pallas_call(
        _loss_body,
        out_shape=jax.ShapeDtypeStruct((1, 1), jnp.float32),
        out_specs=pl.BlockSpec(memory_space=pltpu.SMEM),
    )(scores.reshape(128, 2 * B // 128))
    return out[0, 0]


def _pack_table(x):
    xb = x.astype(jnp.bfloat16)
    xi = lax.bitcast_convert_type(xb.reshape(VOCAB, D2, 2), jnp.int32)
    return xi.reshape(VOCAB // QR, PD)


@jax.jit
def kernel(t_vocab_embs, c_vocab_embs, t, cp, cn):
    t_q = _pack_table(t_vocab_embs)
    c_q = _pack_table(c_vocab_embs)
    t_i = t.astype(jnp.int32)
    cp_i = cp.astype(jnp.int32)
    cn_i = cn.astype(jnp.int32).reshape(B * NNEG)
    scores = _sc_scores(t_q, c_q, t_i, cp_i, cn_i)
    return _tc_loss(scores)


# speedup vs baseline: 2.9909x; 2.9909x over previous
import jax
import jax.numpy as jnp
from jax import lax
from jax.experimental import pallas as pl
from jax.experimental.pallas import tpu as pltpu
from jax.experimental.pallas import tpu_sc as plsc

VOCAB = 1000000
DIM = 64
PD = 128
B = 16384
NNEG = 20

NC = 2
NS = 16
L = 16
NW = NC * NS
BPW = B // NW
CH = 16
NCH = BPW // CH
CNC = CH * NNEG
CN_SPLIT = ((0, 128), (128, 128), (256, 64))


def _fire_a(c_tab, idx_cp, idx_cn, cp_rows, cn_rows, sem, c):
    pltpu.async_copy(c_tab.at[idx_cp.at[pl.ds(c * CH, CH)]], cp_rows, sem)
    for off, n in CN_SPLIT:
        pltpu.async_copy(c_tab.at[idx_cn.at[pl.ds(c * CNC + off, n)]],
                         cn_rows.at[pl.ds(off, n)], sem)


def _drain_a(c_tab, idx_cp, idx_cn, cp_rows, cn_rows, sem, c):
    pltpu.make_async_copy(c_tab.at[idx_cp.at[pl.ds(c * CH, CH)]], cp_rows, sem).wait()
    for off, n in CN_SPLIT:
        pltpu.make_async_copy(c_tab.at[idx_cn.at[pl.ds(c * CNC + off, n)]],
                              cn_rows.at[pl.ds(off, n)], sem).wait()


def _sc_a_body(c_tab, cp_idx, cn_idx, u_out, cp_out,
               idx_cp, idx_cn, cp_rows, cn_rows, u_buf, sem0, sem1):
    wid = lax.axis_index("s") * NC + lax.axis_index("c")
    base = wid * BPW

    pltpu.sync_copy(cp_idx.at[pl.ds(base, BPW)], idx_cp)
    pltpu.sync_copy(cn_idx.at[pl.ds(base * NNEG, BPW * NNEG)], idx_cn)

    lane = lax.iota(jnp.int32, L)
    cn_row = [lane * NNEG + n for n in range(NNEG)]

    def compute(c, k):
        def dot_step(d, _):
            dv = (lane + d) & (DIM - 1)
            cs0 = plsc.load_gather(cn_rows.at[k], [cn_row[0], dv])
            cs1 = plsc.load_gather(cn_rows.at[k], [cn_row[1], dv])
            for n in range(2, NNEG, 2):
                cs0 = cs0 + plsc.load_gather(cn_rows.at[k], [cn_row[n], dv])
                cs1 = cs1 + plsc.load_gather(cn_rows.at[k], [cn_row[n + 1], dv])
            plsc.store_scatter(u_buf.at[k], [lane, dv], cs0 + cs1)
            return 0

        lax.fori_loop(0, DIM, dot_step, 0, unroll=4)
        pltpu.sync_copy(u_buf.at[k],
                        u_out.at[pl.ds(base + c * CH, CH)])
        pltpu.sync_copy(cp_rows.at[k],
                        cp_out.at[pl.ds(base + c * CH, CH)])

    _fire_a(c_tab, idx_cp, idx_cn, cp_rows.at[0], cn_rows.at[0], sem0, 0)

    def loop_body(i, _):
        c0 = 2 * i
        _fire_a(c_tab, idx_cp, idx_cn, cp_rows.at[1], cn_rows.at[1], sem1, c0 + 1)
        _drain_a(c_tab, idx_cp, idx_cn, cp_rows.at[0], cn_rows.at[0], sem0, c0)
        compute(c0, 0)

        @pl.when(c0 + 2 < NCH)
        def _():
            _fire_a(c_tab, idx_cp, idx_cn, cp_rows.at[0], cn_rows.at[0], sem0, c0 + 2)
        _drain_a(c_tab, idx_cp, idx_cn, cp_rows.at[1], cn_rows.at[1], sem1, c0 + 1)
        compute(c0 + 1, 1)
        return 0

    lax.fori_loop(0, NCH // 2, loop_body, 0)


def _sc_part_a(c_tab, cp_idx, cn_idx):
    mesh = plsc.VectorSubcoreMesh(core_axis_name="c", subcore_axis_name="s")
    return pl.kernel(
        _sc_a_body,
        out_type=(jax.ShapeDtypeStruct((B, DIM), jnp.float32),
                  jax.ShapeDtypeStruct((B, PD), jnp.float32)),
        mesh=mesh,
        compiler_params=pltpu.CompilerParams(needs_layout_passes=False),
        scratch_types=[
            pltpu.VMEM((BPW,), jnp.int32),
            pltpu.VMEM((BPW * NNEG,), jnp.int32),
            pltpu.VMEM((2, CH, PD), jnp.float32),
            pltpu.VMEM((2, CNC, PD), jnp.float32),
            pltpu.VMEM((2, CH, DIM), jnp.float32),
            pltpu.SemaphoreType.DMA,
            pltpu.SemaphoreType.DMA,
        ],
    )(c_tab, cp_idx, cn_idx)


def _fire_b(t_tab, idx_t, u_hbm, cp_hbm, base, t_rows, u_rows, cp_rows, sem, c):
    pltpu.async_copy(t_tab.at[idx_t.at[pl.ds(c * CH, CH)]], t_rows, sem)
    pltpu.async_copy(u_hbm.at[pl.ds(base + c * CH, CH)], u_rows, sem)
    pltpu.async_copy(cp_hbm.at[pl.ds(base + c * CH, CH)], cp_rows, sem)


def _drain_b(t_tab, idx_t, u_hbm, cp_hbm, base, t_rows, u_rows, cp_rows, sem, c):
    pltpu.make_async_copy(t_tab.at[idx_t.at[pl.ds(c * CH, CH)]], t_rows, sem).wait()
    pltpu.make_async_copy(u_hbm.at[pl.ds(base + c * CH, CH)], u_rows, sem).wait()
    pltpu.make_async_copy(cp_hbm.at[pl.ds(base + c * CH, CH)], cp_rows, sem).wait()


def _sc_b_body(t_tab, t_idx, u_hbm, cp_hbm, out,
               idx_t, t_rows, u_rows, cp_rows, sp_out, sn_out, sem0, sem1):
    wid = lax.axis_index("s") * NC + lax.axis_index("c")
    base = wid * BPW

    pltpu.sync_copy(t_idx.at[pl.ds(base, BPW)], idx_t)

    lane = lax.iota(jnp.int32, L)

    def compute(c, k):
        def dot_step(d, carry):
            s_p, s_n = carry
            dv = (lane + d) & (DIM - 1)
            td = plsc.load_gather(t_rows.at[k], [lane, dv])
            cpd = plsc.load_gather(cp_rows.at[k], [lane, dv])
            ud = plsc.load_gather(u_rows.at[k], [lane, dv])
            return s_p + td * cpd, s_n + td * ud

        zero = jnp.zeros((L,), jnp.float32)
        s_p, s_n = lax.fori_loop(0, DIM, dot_step, (zero, zero), unroll=8)
        sp_out[pl.ds(c * CH, L)] = -s_p
        sn_out[pl.ds(c * CH, L)] = s_n

    args = (t_tab, idx_t, u_hbm, cp_hbm, base)
    _fire_b(*args, t_rows.at[0], u_rows.at[0], cp_rows.at[0], sem0, 0)

    def loop_body(i, _):
        c0 = 2 * i
        _fire_b(*args, t_rows.at[1], u_rows.at[1], cp_rows.at[1], sem1, c0 + 1)
        _drain_b(*args, t_rows.at[0], u_rows.at[0], cp_rows.at[0], sem0, c0)
        compute(c0, 0)

        @pl.when(c0 + 2 < NCH)
        def _():
            _fire_b(*args, t_rows.at[0], u_rows.at[0], cp_rows.at[0], sem0, c0 + 2)
        _drain_b(*args, t_rows.at[1], u_rows.at[1], cp_rows.at[1], sem1, c0 + 1)
        compute(c0 + 1, 1)
        return 0

    lax.fori_loop(0, NCH // 2, loop_body, 0)

    pltpu.sync_copy(sp_out, out.at[pl.ds(base, BPW)])
    pltpu.sync_copy(sn_out, out.at[pl.ds(B + base, BPW)])


def _sc_part_b(t_tab, t_idx, u_hbm, cp_hbm):
    mesh = plsc.VectorSubcoreMesh(core_axis_name="c", subcore_axis_name="s")
    return pl.kernel(
        _sc_b_body,
        out_type=jax.ShapeDtypeStruct((2 * B,), jnp.float32),
        mesh=mesh,
        compiler_params=pltpu.CompilerParams(needs_layout_passes=False),
        scratch_types=[
            pltpu.VMEM((BPW,), jnp.int32),
            pltpu.VMEM((2, CH, PD), jnp.float32),
            pltpu.VMEM((2, CH, DIM), jnp.float32),
            pltpu.VMEM((2, CH, PD), jnp.float32),
            pltpu.VMEM((BPW,), jnp.float32),
            pltpu.VMEM((BPW,), jnp.float32),
            pltpu.SemaphoreType.DMA,
            pltpu.SemaphoreType.DMA,
        ],
    )(t_tab, t_idx, u_hbm, cp_hbm)


def _loss_body(s_ref, o_ref):
    x = s_ref[...]
    z = jnp.minimum(x, 0.0) - jnp.log(1.0 + jnp.exp(-jnp.abs(x)))
    o_ref[0, 0] = jnp.sum(z) * (1.0 / B)


def _tc_loss(scores):
    out = pl.pallas_call(
        _loss_body,
        out_shape=jax.ShapeDtypeStruct((1, 1), jnp.float32),
        out_specs=pl.BlockSpec(memory_space=pltpu.SMEM),
    )(scores.reshape(128, 2 * B // 128))
    return out[0, 0]


@jax.jit
def kernel(t_vocab_embs, c_vocab_embs, t, cp, cn):
    t_pad = jnp.pad(t_vocab_embs, ((0, 0), (0, PD - DIM)))
    c_pad = jnp.pad(c_vocab_embs, ((0, 0), (0, PD - DIM)))
    t_i = t.astype(jnp.int32)
    cp_i = cp.astype(jnp.int32)
    cn_i = cn.astype(jnp.int32).reshape(B * NNEG)
    u_stage, cp_stage = _sc_part_a(c_pad, cp_i, cn_i)
    scores = _sc_part_b(t_pad, t_i, u_stage, cp_stage)
    return _tc_loss(scores)
